# baseline (speedup 1.0000x reference)
import jax
import jax.numpy as jnp
from jax.experimental import pallas as pl
from jax.experimental.pallas import tpu as pltpu

_B, _MD, _CD, _H, _E = 4096, 138, 11, 512, 8
_ID = _MD + _CD
_IDP = 160
_BB = 512
_NC = _B // _BB
_EH = _E * _H
_EIDP = _E * _IDP


def _elu_bf(v):
    return jnp.where(v > 0, v, jnp.exp(jnp.minimum(v, 0.0)) - 1.0
                     ).astype(jnp.bfloat16)


def _moe_body(motion_ref, command_ref, gW1_ref, gb1_ref, gW2_ref, gb2_ref,
              W0_ref, b0_ref, W1_ref, b1_ref, W2_ref, b2_ref,
              W3_ref, b3_ref, out_ref,
              g1s, g1b, g2s, g2b, Wc0, b0s, Wc1, b1s, Wc2, b2s, Wc3, b3s,
              stgA, stgB, stg3, xts, cbs, y0, rs, sem1, sem2, sem3):
    f32 = jnp.float32
    bf = jnp.bfloat16

    cpW1 = pltpu.make_async_copy(W1_ref, stgA, sem1)
    cpW2 = pltpu.make_async_copy(W2_ref, stgB, sem2)
    cpW3 = pltpu.make_async_copy(W3_ref, stg3, sem3)
    cpW1.start()
    cpW2.start()
    cpW3.start()

    g1s[...] = gW1_ref[...].T.astype(bf)
    g1b[...] = gb1_ref[...].T
    g2s[...] = gW2_ref[...].T.astype(bf)
    g2b[...] = gb2_ref[...].T
    b0s[...] = b0_ref[...].T.astype(bf)
    b1s[...] = b1_ref[...].T.astype(bf)
    b2s[...] = b2_ref[...].T.astype(bf)
    b3s[...] = b3_ref[...].T.astype(bf)
    Wc0[...] = jnp.zeros((_H, _EIDP), bf)
    rs[0:_EIDP, :] = jnp.zeros((_EIDP, _BB), bf)
    for e in range(_E):
        Wc0[:, e * _IDP:e * _IDP + _ID] = W0_ref[e].astype(bf)

    def gating(j, carry):
        sl = pl.ds(j * _BB, _BB)
        xt = jnp.concatenate([motion_ref[sl, :].T, command_ref[sl, :].T],
                             axis=0).astype(bf)
        xts[:, sl] = xt
        h = jnp.dot(g1s[...], xt, preferred_element_type=f32) + g1b[...]
        h = _elu_bf(h)
        logits = (jnp.dot(g2s[...], h, preferred_element_type=f32)
                  + g2b[...])
        mx = jnp.max(logits, axis=0, keepdims=True)
        p = jnp.exp(logits - mx)
        cbs[:, sl] = (p / jnp.sum(p, axis=0, keepdims=True)).astype(bf)
        return carry

    jax.lax.fori_loop(0, _NC, gating, 0, unroll=2)

    def layer0(j, carry):
        sl = pl.ds(j * _BB, _BB)
        xt = xts[:, sl]
        cb = cbs[:, sl]
        for e in range(_E):
            rs[e * _IDP:e * _IDP + _ID, :] = xt * cb[e:e + 1, :]
        acc = jnp.dot(Wc0[...], rs[0:_EIDP, :], preferred_element_type=f32)
        acc = acc + jnp.dot(b0s[...], cb, preferred_element_type=f32)
        y0[:, sl] = _elu_bf(acc)
        return carry

    jax.lax.fori_loop(0, _NC, layer0, 0, unroll=2)

    def mk_layer(Wc, bs, act, last):
        def body(j, carry):
            sl = pl.ds(j * _BB, _BB)
            cb = cbs[:, sl]
            inp = y0[:, sl]
            for e in range(_E):
                rs[e * _H:(e + 1) * _H, :] = inp * cb[e:e + 1, :]
            acc = jnp.dot(Wc[...], rs[...], preferred_element_type=f32)
            acc = acc + jnp.dot(bs[...], cb, preferred_element_type=f32)
            if last:
                out_ref[sl, :] = acc.T
            else:
                y0[:, sl] = _elu_bf(acc)
            return carry
        return body

    cpW1.wait()
    for e in range(_E):
        Wc1[:, e * _H:(e + 1) * _H] = stgA[e].astype(bf)
    jax.lax.fori_loop(0, _NC, mk_layer(Wc1, b1s, True, False), 0, unroll=2)

    cpW2.wait()
    for e in range(_E):
        Wc2[:, e * _H:(e + 1) * _H] = stgB[e].astype(bf)
    jax.lax.fori_loop(0, _NC, mk_layer(Wc2, b2s, True, False), 0, unroll=2)

    cpW3.wait()
    for e in range(_E):
        Wc3[:, e * _H:(e + 1) * _H] = stg3[e].astype(bf)
    jax.lax.fori_loop(0, _NC, mk_layer(Wc3, b3s, False, True), 0, unroll=2)


def kernel(motion, command, gW1, gb1, gW2, gb2, W0, b0, W1, b1, W2, b2, W3, b3):
    bf = jnp.bfloat16
    f32 = jnp.float32
    vmem = pl.BlockSpec(memory_space=pltpu.MemorySpace.VMEM)
    hbm = pl.BlockSpec(memory_space=pl.MemorySpace.ANY)
    in_specs = [vmem, vmem, vmem, vmem, vmem, vmem,
                vmem, vmem,
                hbm, vmem,
                hbm, vmem,
                hbm, vmem]
    scratch_shapes = [
        pltpu.VMEM((_H, _ID), bf),
        pltpu.VMEM((_H, 1), f32),
        pltpu.VMEM((_E, _H), bf),
        pltpu.VMEM((_E, 1), f32),
        pltpu.VMEM((_H, _EIDP), bf),
        pltpu.VMEM((_H, _E), bf),
        pltpu.VMEM((_H, _EH), bf),
        pltpu.VMEM((_H, _E), bf),
        pltpu.VMEM((_H, _EH), bf),
        pltpu.VMEM((_H, _E), bf),
        pltpu.VMEM((_MD, _EH), bf),
        pltpu.VMEM((_MD, _E), bf),
        pltpu.VMEM((_E, _H, _H), f32),
        pltpu.VMEM((_E, _H, _H), f32),
        pltpu.VMEM((_E, _MD, _H), f32),
        pltpu.VMEM((_ID, _B), bf),
        pltpu.VMEM((_E, _B), bf),
        pltpu.VMEM((_H, _B), bf),
        pltpu.VMEM((_EH, _BB), bf),
        pltpu.SemaphoreType.DMA,
        pltpu.SemaphoreType.DMA,
        pltpu.SemaphoreType.DMA,
    ]
    out = pl.pallas_call(
        _moe_body,
        out_shape=jax.ShapeDtypeStruct((_B, _MD), jnp.float32),
        in_specs=in_specs,
        out_specs=pl.BlockSpec(memory_space=pltpu.MemorySpace.VMEM),
        scratch_shapes=scratch_shapes,
    )(motion, command, gW1, gb1.reshape(1, _H), gW2, gb2.reshape(1, _E),
      W0, b0, W1, b1, W2, b2, W3, b3)
    return out
